# baseline (speedup 1.0000x reference)
import jax
import jax.numpy as jnp
from jax.experimental import pallas as pl
from jax.experimental.pallas import tpu as pltpu

_B = 3
_S = 2048
_E = 1024
_HD = 64
_BS = 16
_SCALE = _HD ** -0.5


def _paged_attn_kernel(x_hbm, x16_ref, wq_ref, wk_ref, wv_ref, out_ref,
                       xbuf, sem):
    def copy(b):
        return pltpu.make_async_copy(
            x_hbm.at[pl.ds(b * _S, _S), :], xbuf.at[b % 3], sem.at[b % 3])

    copy(0).start()
    copy(1).start()

    dn_nt = (((1,), (1,)), ((), ()))
    dn_nn = (((1,), (0,)), ((), ()))

    for b in range(_B):
        x16 = x16_ref[b]
        k16 = jax.lax.dot_general(x16, wk_ref[:, :], dn_nt,
                                  preferred_element_type=jnp.float32)
        v16 = jax.lax.dot_general(x16, wv_ref[:, :], dn_nt,
                                  preferred_element_type=jnp.float32)
        a = jax.lax.dot_general(k16 * _SCALE, wq_ref[:, :], dn_nn,
                                preferred_element_type=jnp.float32)
        v_aug = jnp.concatenate(
            [v16, jnp.ones((_BS, 1), jnp.float32)], axis=1)

        copy(b).wait()
        if b + 2 < _B:
            copy(b + 2).start()
        x_tile = xbuf[b % 3]

        s_t = jax.lax.dot_general(a, x_tile, dn_nt,
                                  preferred_element_type=jnp.float32)

        row = jax.lax.broadcasted_iota(jnp.int32, (_BS, _S), 1)
        m = jax.lax.broadcasted_iota(jnp.int32, (_BS, _S), 0)
        d = row >> 4
        r = row & (_BS - 1)
        cnt = d.astype(jnp.float32) + (m <= r).astype(jnp.float32)

        smax = jnp.max(s_t, axis=0, keepdims=True)
        w = cnt * jnp.exp(s_t - smax)

        out_aug = jax.lax.dot_general(w, v_aug,
                                      (((0,), (0,)), ((), ())),
                                      preferred_element_type=jnp.float32)
        out_ref[pl.ds(b * _S, _S), :] = out_aug[:, :_HD] / out_aug[:, _HD:]


@jax.jit
def kernel(x, Wq, Wk, Wv):
    xf = x.reshape(_B * _S, _E)
    out = pl.pallas_call(
        _paged_attn_kernel,
        in_specs=[
            pl.BlockSpec(memory_space=pltpu.MemorySpace.HBM),
            pl.BlockSpec((_B, _BS, _E), lambda: (0, 0, 0)),
            pl.BlockSpec((_HD, _E), lambda: (0, 0)),
            pl.BlockSpec((_HD, _E), lambda: (0, 0)),
            pl.BlockSpec((_HD, _E), lambda: (0, 0)),
        ],
        out_specs=pl.BlockSpec((_B * _S, _HD), lambda: (0, 0)),
        out_shape=jax.ShapeDtypeStruct((_B * _S, _HD), jnp.float32),
        scratch_shapes=[
            pltpu.VMEM((3, _S, _E), jnp.float32),
            pltpu.SemaphoreType.DMA((3,)),
        ],
    )(xf, x[:, :_BS, :], Wq, Wk, Wv)
    return out.reshape(_B, _S, _HD)


# speedup vs baseline: 1.0096x; 1.0096x over previous
import jax
import jax.numpy as jnp
from jax.experimental import pallas as pl
from jax.experimental.pallas import tpu as pltpu

_B = 3
_S = 2048
_E = 1024
_HD = 64
_BS = 16
_SCALE = _HD ** -0.5


def _paged_attn_kernel(x_hbm, x16_ref, wq_ref, wk_ref, wv_ref, out_ref,
                       xbuf, sem):
    def copy(b):
        return pltpu.make_async_copy(
            x_hbm.at[pl.ds(b * _S, _S), :], xbuf.at[b % 3], sem.at[b % 3])

    copy(0).start()
    copy(1).start()
    copy(2).start()

    dn_nt = (((1,), (1,)), ((), ()))
    dn_nn = (((1,), (0,)), ((), ()))

    for b in range(_B):
        x16 = x16_ref[b]
        k16 = jax.lax.dot_general(x16, wk_ref[:, :], dn_nt,
                                  preferred_element_type=jnp.float32)
        v16 = jax.lax.dot_general(x16, wv_ref[:, :], dn_nt,
                                  preferred_element_type=jnp.float32)
        a = jax.lax.dot_general(k16 * _SCALE, wq_ref[:, :], dn_nn,
                                preferred_element_type=jnp.float32)

        copy(b).wait()
        x_tile = xbuf[b % 3]

        s_t = jax.lax.dot_general(a, x_tile, dn_nt,
                                  preferred_element_type=jnp.float32)

        row = jax.lax.broadcasted_iota(jnp.int32, (_BS, _S), 1)
        m = jax.lax.broadcasted_iota(jnp.int32, (_BS, _S), 0)
        d = row >> 4
        r = row & (_BS - 1)
        cnt = d.astype(jnp.float32) + (m <= r).astype(jnp.float32)

        smax = jnp.max(s_t, axis=0, keepdims=True)
        w = cnt * jnp.exp(s_t - smax)
        w = w / jnp.sum(w, axis=0, keepdims=True)

        out = jax.lax.dot_general(w, v16,
                                  (((0,), (0,)), ((), ())),
                                  preferred_element_type=jnp.float32)
        out_ref[pl.ds(b * _S, _S), :] = out


@jax.jit
def kernel(x, Wq, Wk, Wv):
    xf = x.reshape(_B * _S, _E)
    out = pl.pallas_call(
        _paged_attn_kernel,
        in_specs=[
            pl.BlockSpec(memory_space=pltpu.MemorySpace.HBM),
            pl.BlockSpec((_B, _BS, _E), lambda: (0, 0, 0)),
            pl.BlockSpec((_HD, _E), lambda: (0, 0)),
            pl.BlockSpec((_HD, _E), lambda: (0, 0)),
            pl.BlockSpec((_HD, _E), lambda: (0, 0)),
        ],
        out_specs=pl.BlockSpec((_B * _S, _HD), lambda: (0, 0)),
        out_shape=jax.ShapeDtypeStruct((_B * _S, _HD), jnp.float32),
        scratch_shapes=[
            pltpu.VMEM((3, _S, _E), jnp.float32),
            pltpu.SemaphoreType.DMA((3,)),
        ],
    )(xf, x[:, :_BS, :], Wq, Wk, Wv)
    return out.reshape(_B, _S, _HD)


# speedup vs baseline: 1.0853x; 1.0750x over previous
import jax
import jax.numpy as jnp
from jax.experimental import pallas as pl
from jax.experimental.pallas import tpu as pltpu

_B = 3
_S = 2048
_E = 1024
_HD = 64
_BS = 16
_SCALE = _HD ** -0.5


def _paged_attn_kernel(x_hbm, wq_ref, wk_ref, wv_ref, out_ref,
                       xbuf, x16buf, sem, sem16):
    def copy(b):
        return pltpu.make_async_copy(
            x_hbm.at[pl.ds(b * _S, _S), :], xbuf.at[b % 3], sem.at[b % 3])

    def copy16(b):
        return pltpu.make_async_copy(
            x_hbm.at[pl.ds(b * _S, _BS), :], x16buf.at[b], sem16.at[b])

    copy16(0).start()
    copy16(1).start()
    copy16(2).start()
    copy(0).start()
    copy(1).start()
    copy(2).start()

    dn_nt = (((1,), (1,)), ((), ()))
    dn_nn = (((1,), (0,)), ((), ()))

    for b in range(_B):
        copy16(b).wait()
        x16 = x16buf[b]
        k16 = jax.lax.dot_general(x16, wk_ref[:, :], dn_nt,
                                  preferred_element_type=jnp.float32)
        v16 = jax.lax.dot_general(x16, wv_ref[:, :], dn_nt,
                                  preferred_element_type=jnp.float32)
        a = jax.lax.dot_general(k16 * _SCALE, wq_ref[:, :], dn_nn,
                                preferred_element_type=jnp.float32)

        copy(b).wait()
        x_tile = xbuf[b % 3]

        s_t = jax.lax.dot_general(a, x_tile, dn_nt,
                                  preferred_element_type=jnp.float32)

        row = jax.lax.broadcasted_iota(jnp.int32, (_BS, _S), 1)
        m = jax.lax.broadcasted_iota(jnp.int32, (_BS, _S), 0)
        d = row >> 4
        r = row & (_BS - 1)
        cnt = d.astype(jnp.float32) + (m <= r).astype(jnp.float32)

        smax = jnp.max(s_t, axis=0, keepdims=True)
        w = cnt * jnp.exp(s_t - smax)
        w = w / jnp.sum(w, axis=0, keepdims=True)

        out = jax.lax.dot_general(w, v16,
                                  (((0,), (0,)), ((), ())),
                                  preferred_element_type=jnp.float32)
        out_ref[pl.ds(b * _S, _S), :] = out


@jax.jit
def kernel(x, Wq, Wk, Wv):
    xf = x.reshape(_B * _S, _E)
    out = pl.pallas_call(
        _paged_attn_kernel,
        in_specs=[
            pl.BlockSpec(memory_space=pltpu.MemorySpace.HBM),
            pl.BlockSpec((_HD, _E), lambda: (0, 0)),
            pl.BlockSpec((_HD, _E), lambda: (0, 0)),
            pl.BlockSpec((_HD, _E), lambda: (0, 0)),
        ],
        out_specs=pl.BlockSpec((_B * _S, _HD), lambda: (0, 0)),
        out_shape=jax.ShapeDtypeStruct((_B * _S, _HD), jnp.float32),
        scratch_shapes=[
            pltpu.VMEM((3, _S, _E), jnp.float32),
            pltpu.VMEM((_B, _BS, _E), jnp.float32),
            pltpu.SemaphoreType.DMA((3,)),
            pltpu.SemaphoreType.DMA((_B,)),
        ],
    )(xf, Wq, Wk, Wv)
    return out.reshape(_B, _S, _HD)


# speedup vs baseline: 1.1285x; 1.0398x over previous
import jax
import jax.numpy as jnp
from jax.experimental import pallas as pl
from jax.experimental.pallas import tpu as pltpu

_B = 3
_S = 2048
_E = 1024
_HD = 64
_BS = 16
_CHUNK = 1024
_NBUF = 4
_SCALE = _HD ** -0.5


def _paged_attn_kernel(x_hbm, wq_ref, wk_ref, wv_ref, out_ref,
                       xbuf, x16buf, sem, sem16):
    nchunks = _B * _S // _CHUNK
    chunks_per_req = _S // _CHUNK

    def copy(c):
        return pltpu.make_async_copy(
            x_hbm.at[pl.ds(c * _CHUNK, _CHUNK), :], xbuf.at[c % _NBUF],
            sem.at[c % _NBUF])

    def copy16(b):
        return pltpu.make_async_copy(
            x_hbm.at[pl.ds(b * _S, _BS), :], x16buf.at[b], sem16.at[b])

    for b in range(_B):
        copy16(b).start()
    for c in range(_NBUF):
        copy(c).start()

    dn_nt = (((1,), (1,)), ((), ()))
    dn_nn = (((1,), (0,)), ((), ()))

    a_req = [None] * _B
    v_req = [None] * _B
    for c in range(nchunks):
        b = c // chunks_per_req
        if c % chunks_per_req == 0:
            copy16(b).wait()
            x16 = x16buf[b]
            k16 = jax.lax.dot_general(x16, wk_ref[:, :], dn_nt,
                                      preferred_element_type=jnp.float32)
            v_req[b] = jax.lax.dot_general(x16, wv_ref[:, :], dn_nt,
                                           preferred_element_type=jnp.float32)
            a_req[b] = jax.lax.dot_general(k16 * _SCALE, wq_ref[:, :], dn_nn,
                                           preferred_element_type=jnp.float32)

        copy(c).wait()
        x_tile = xbuf[c % _NBUF]

        s_t = jax.lax.dot_general(a_req[b], x_tile, dn_nt,
                                  preferred_element_type=jnp.float32)

        row = jax.lax.broadcasted_iota(jnp.int32, (_BS, _CHUNK), 1)
        m = jax.lax.broadcasted_iota(jnp.int32, (_BS, _CHUNK), 0)
        d = (c % chunks_per_req) * (_CHUNK // _BS) + (row >> 4)
        r = row & (_BS - 1)
        cnt = d.astype(jnp.float32) + (m <= r).astype(jnp.float32)

        smax = jnp.max(s_t, axis=0, keepdims=True)
        w = cnt * jnp.exp(s_t - smax)
        w = w / jnp.sum(w, axis=0, keepdims=True)

        out = jax.lax.dot_general(w, v_req[b],
                                  (((0,), (0,)), ((), ())),
                                  preferred_element_type=jnp.float32)
        out_ref[pl.ds(c * _CHUNK, _CHUNK), :] = out

        if c + _NBUF < nchunks:
            copy(c + _NBUF).start()


@jax.jit
def kernel(x, Wq, Wk, Wv):
    xf = x.reshape(_B * _S, _E)
    out = pl.pallas_call(
        _paged_attn_kernel,
        in_specs=[
            pl.BlockSpec(memory_space=pltpu.MemorySpace.HBM),
            pl.BlockSpec((_HD, _E), lambda: (0, 0)),
            pl.BlockSpec((_HD, _E), lambda: (0, 0)),
            pl.BlockSpec((_HD, _E), lambda: (0, 0)),
        ],
        out_specs=pl.BlockSpec((_B * _S, _HD), lambda: (0, 0)),
        out_shape=jax.ShapeDtypeStruct((_B * _S, _HD), jnp.float32),
        scratch_shapes=[
            pltpu.VMEM((_NBUF, _CHUNK, _E), jnp.float32),
            pltpu.VMEM((_B, _BS, _E), jnp.float32),
            pltpu.SemaphoreType.DMA((_NBUF,)),
            pltpu.SemaphoreType.DMA((_B,)),
        ],
    )(xf, Wq, Wk, Wv)
    return out.reshape(_B, _S, _HD)
